# Initial kernel scaffold; baseline (speedup 1.0000x reference)
import jax
import jax.numpy as jnp
from jax.experimental import pallas as pl


def kernel(inputs, embeddings):
    raise NotImplementedError("write your pallas kernel here")


# speedup vs baseline: 8.3030x; 8.3030x over previous
import jax
import jax.numpy as jnp
from jax.experimental import pallas as pl
from jax.experimental.pallas import tpu as pltpu

MAXSPAN = 255
QLEN = 2048
KLEN = 2048
DEPTH = 64
EXT = 4096
LO_PAD = 1793
HI_START = 2302
NBUF = 8


def _build_ext_kernel(emb_ref, ext_ref):
    ext_ref[0:LO_PAD, :] = jnp.broadcast_to(emb_ref[0:1, :], (LO_PAD, DEPTH))
    ext_ref[LO_PAD:HI_START, :] = emb_ref[:, :]
    ext_ref[HI_START:EXT, :] = jnp.broadcast_to(
        emb_ref[508:509, :], (EXT - HI_START, DEPTH))


def _expand_kernel(ext_ref, out_ref, sems):
    def issue(i):
        return pltpu.make_async_copy(
            ext_ref.at[pl.ds(QLEN - 1 - i, KLEN), :],
            out_ref.at[i],
            sems.at[i % NBUF])

    def loop(i, _):
        @pl.when(i >= NBUF)
        def _():
            issue(i - NBUF).wait()
        issue(i).start()
        return ()

    jax.lax.fori_loop(0, QLEN, loop, ())

    def tail(i, _):
        issue(QLEN - NBUF + i).wait()
        return ()

    jax.lax.fori_loop(0, NBUF, tail, ())


def kernel(inputs, embeddings):
    del inputs
    ext = pl.pallas_call(
        _build_ext_kernel,
        out_shape=jax.ShapeDtypeStruct((EXT, DEPTH), jnp.float32),
    )(embeddings)
    out = pl.pallas_call(
        _expand_kernel,
        in_specs=[pl.BlockSpec(memory_space=pltpu.MemorySpace.VMEM)],
        out_specs=pl.BlockSpec(memory_space=pl.ANY),
        out_shape=jax.ShapeDtypeStruct((QLEN, KLEN, DEPTH), jnp.float32),
        scratch_shapes=[pltpu.SemaphoreType.DMA((NBUF,))],
    )(ext)
    return out
